# baseline (speedup 1.0000x reference)
import jax
import jax.numpy as jnp
from jax import lax
from jax.experimental import pallas as pl
from jax.experimental.pallas import tpu as pltpu
from jax.experimental.pallas import tpu_sc as plsc

_MAX_SHIFT = 7
_NUM_XS = 2 * _MAX_SHIFT + 1
_N = 320000
_D = 128
_NW = 32
_BPW = _N // _NW
_CH = 80
_NCH = _BPW // _CH
_GRP = 16


def _body(x_ref, emb_ref, out_ref, xv, idx_v, rows, gsem):
    wid = lax.axis_index("s") * 2 + lax.axis_index("c")
    ebase = wid * _BPW

    pltpu.sync_copy(x_ref.at[pl.ds(ebase * 3, _BPW * 3)], xv)

    lanes3 = lax.iota(jnp.int32, _GRP) * 3

    def idx_body(g, carry):
        b = g * (3 * _GRP)
        outward = plsc.load_gather(xv, [lanes3 + b])
        xs = plsc.load_gather(xv, [lanes3 + (b + 1)])
        ys = plsc.load_gather(xv, [lanes3 + (b + 2)])
        idx = 2 * ((xs + _MAX_SHIFT) * _NUM_XS + (ys + _MAX_SHIFT)) + outward
        idx_v[pl.ds(g * _GRP, _GRP)] = idx
        return carry

    lax.fori_loop(0, _BPW // _GRP, idx_body, 0)

    def ch_body(c, carry):
        idx_sl = idx_v.at[pl.ds(c * _CH, _CH)]
        pltpu.async_copy(emb_ref.at[idx_sl], rows, gsem).wait()
        pltpu.sync_copy(rows, out_ref.at[pl.ds(ebase + c * _CH, _CH)])
        return carry

    lax.fori_loop(0, _NCH, ch_body, 0)


def kernel(x, emb):
    mesh = plsc.VectorSubcoreMesh(core_axis_name="c", subcore_axis_name="s")
    f = pl.kernel(
        _body,
        out_type=jax.ShapeDtypeStruct((_N, _D), jnp.float32),
        mesh=mesh,
        compiler_params=pltpu.CompilerParams(needs_layout_passes=False),
        scratch_types=[
            pltpu.VMEM((_BPW * 3,), jnp.int32),
            pltpu.VMEM((_BPW,), jnp.int32),
            pltpu.VMEM((_CH, _D), jnp.float32),
            pltpu.SemaphoreType.DMA,
        ],
    )
    return f(x.reshape(-1), emb)


# speedup vs baseline: 1.0059x; 1.0059x over previous
import jax
import jax.numpy as jnp
from jax import lax
from jax.experimental import pallas as pl
from jax.experimental.pallas import tpu as pltpu
from jax.experimental.pallas import tpu_sc as plsc

_MAX_SHIFT = 7
_NUM_XS = 2 * _MAX_SHIFT + 1
_N = 320000
_D = 128
_NW = 32
_BPW = _N // _NW
_CH = 80
_NCH = _BPW // _CH
_GRP = 16
_NBUF = 5


def _body(x_ref, emb_ref, out_ref, xv, idx_v, rows, gsem, ssem):
    wid = lax.axis_index("s") * 2 + lax.axis_index("c")
    ebase = wid * _BPW

    pltpu.sync_copy(x_ref.at[pl.ds(ebase * 3, _BPW * 3)], xv)

    lanes3 = lax.iota(jnp.int32, _GRP) * 3

    def idx_body(g, carry):
        b = g * (3 * _GRP)
        outward = plsc.load_gather(xv, [lanes3 + b])
        xs = plsc.load_gather(xv, [lanes3 + (b + 1)])
        ys = plsc.load_gather(xv, [lanes3 + (b + 2)])
        idx = 2 * ((xs + _MAX_SHIFT) * _NUM_XS + (ys + _MAX_SHIFT)) + outward
        idx_v[pl.ds(g * _GRP, _GRP)] = idx
        return carry

    lax.fori_loop(0, _BPW // _GRP, idx_body, 0)

    def start_gather(b, c):
        idx_sl = idx_v.at[pl.ds(c * _CH, _CH)]
        pltpu.async_copy(emb_ref.at[idx_sl], rows.at[b], gsem.at[b])

    def wait_gather(b):
        pltpu.make_async_copy(
            out_ref.at[pl.ds(0, _CH)], rows.at[b], gsem.at[b]).wait()

    def start_store(b, c):
        pltpu.async_copy(
            rows.at[b], out_ref.at[pl.ds(ebase + c * _CH, _CH)], ssem.at[b])

    def wait_store(b):
        pltpu.make_async_copy(
            rows.at[b], out_ref.at[pl.ds(0, _CH)], ssem.at[b]).wait()

    for b in range(_NBUF):
        start_gather(b, b)

    def ch_body(p, carry):
        for b in range(_NBUF):
            c = p * _NBUF + b
            wait_gather(b)
            start_store(b, c)

            @pl.when(c + _NBUF < _NCH)
            def _():
                wait_store(b)
                start_gather(b, c + _NBUF)

        return carry

    lax.fori_loop(0, _NCH // _NBUF, ch_body, 0)
    for b in range(_NBUF):
        wait_store(b)


def kernel(x, emb):
    mesh = plsc.VectorSubcoreMesh(core_axis_name="c", subcore_axis_name="s")
    f = pl.kernel(
        _body,
        out_type=jax.ShapeDtypeStruct((_N, _D), jnp.float32),
        mesh=mesh,
        compiler_params=pltpu.CompilerParams(needs_layout_passes=False),
        scratch_types=[
            pltpu.VMEM((_BPW * 3,), jnp.int32),
            pltpu.VMEM((_BPW,), jnp.int32),
            pltpu.VMEM((_NBUF, _CH, _D), jnp.float32),
            pltpu.SemaphoreType.DMA((_NBUF,)),
            pltpu.SemaphoreType.DMA((_NBUF,)),
        ],
    )
    return f(x.reshape(-1), emb)


# speedup vs baseline: 7.3422x; 7.2991x over previous
import jax
import jax.numpy as jnp
from jax import lax
from jax.experimental import pallas as pl
from jax.experimental.pallas import tpu as pltpu
from jax.experimental.pallas import tpu_sc as plsc

_MAX_SHIFT = 7
_NUM_XS = 2 * _MAX_SHIFT + 1
_N = 320000
_D = 128
_NW = 32
_BPW = _N // _NW
_CH = 80
_NCH = _BPW // _CH
_GRP = 16
_NBUF = 5
_NEMB = 450


def _body(x_ref, emb_ref, out_ref, xv, idx_v, table_v, rows, gsem, ssem):
    wid = lax.axis_index("s") * 2 + lax.axis_index("c")
    ebase = wid * _BPW

    @pl.when(lax.axis_index("s") == 0)
    def _():
        pltpu.sync_copy(emb_ref, table_v)

    plsc.subcore_barrier()
    pltpu.sync_copy(x_ref.at[pl.ds(ebase * 3, _BPW * 3)], xv)

    lanes3 = lax.iota(jnp.int32, _GRP) * 3

    def idx_body(g, carry):
        b = g * (3 * _GRP)
        outward = plsc.load_gather(xv, [lanes3 + b])
        xs = plsc.load_gather(xv, [lanes3 + (b + 1)])
        ys = plsc.load_gather(xv, [lanes3 + (b + 2)])
        idx = 2 * ((xs + _MAX_SHIFT) * _NUM_XS + (ys + _MAX_SHIFT)) + outward
        idx_v[pl.ds(g * _GRP, _GRP)] = idx
        return carry

    lax.fori_loop(0, _BPW // _GRP, idx_body, 0)

    def start_gather(b, c):
        idx_sl = idx_v.at[pl.ds(c * _CH, _CH)]
        pltpu.async_copy(table_v.at[idx_sl], rows.at[b], gsem.at[b])

    def wait_gather(b):
        pltpu.make_async_copy(
            out_ref.at[pl.ds(0, _CH)], rows.at[b], gsem.at[b]).wait()

    def start_store(b, c):
        pltpu.async_copy(
            rows.at[b], out_ref.at[pl.ds(ebase + c * _CH, _CH)], ssem.at[b])

    def wait_store(b):
        pltpu.make_async_copy(
            rows.at[b], out_ref.at[pl.ds(0, _CH)], ssem.at[b]).wait()

    def ch_body(p, carry):
        for b in range(_NBUF):
            c = p * _NBUF + b
            wait_gather(b)
            start_store(b, c)

            @pl.when(c + _NBUF < _NCH)
            def _():
                wait_store(b)
                start_gather(b, c + _NBUF)

        return carry

    for b in range(_NBUF):
        start_gather(b, b)
    lax.fori_loop(0, _NCH // _NBUF, ch_body, 0)
    for b in range(_NBUF):
        wait_store(b)


def kernel(x, emb):
    mesh = plsc.VectorSubcoreMesh(core_axis_name="c", subcore_axis_name="s")
    f = pl.kernel(
        _body,
        out_type=jax.ShapeDtypeStruct((_N, _D), jnp.float32),
        mesh=mesh,
        compiler_params=pltpu.CompilerParams(needs_layout_passes=False),
        scratch_types=[
            pltpu.VMEM((_BPW * 3,), jnp.int32),
            pltpu.VMEM((_BPW,), jnp.int32),
            pltpu.VMEM_SHARED((_NEMB, _D), jnp.float32),
            pltpu.VMEM((_NBUF, _CH, _D), jnp.float32),
            pltpu.SemaphoreType.DMA((_NBUF,)),
            pltpu.SemaphoreType.DMA((_NBUF,)),
        ],
    )
    return f(x.reshape(-1), emb)


# speedup vs baseline: 7.4309x; 1.0121x over previous
import jax
import jax.numpy as jnp
from jax import lax
from jax.experimental import pallas as pl
from jax.experimental.pallas import tpu as pltpu
from jax.experimental.pallas import tpu_sc as plsc

_MAX_SHIFT = 7
_NUM_XS = 2 * _MAX_SHIFT + 1
_N = 320000
_D = 128
_NW = 32
_BPW = _N // _NW
_CH = 80
_NCH = _BPW // _CH
_GRP = 16
_NBUF = 5
_NEMB = 450


def _body(x_ref, emb_ref, out_ref, xv, idx_v, table_v, rows, gsem, ssem):
    wid = lax.axis_index("s") * 2 + lax.axis_index("c")
    ebase = wid * _BPW

    @pl.when(lax.axis_index("s") == 0)
    def _():
        pltpu.sync_copy(emb_ref, table_v)

    plsc.subcore_barrier()
    pltpu.sync_copy(x_ref.at[pl.ds(ebase * 3, _BPW * 3)], xv)

    lanes3 = lax.iota(jnp.int32, _GRP) * 3

    def compute_idx_chunk(c):
        for g in range(_CH // _GRP):
            b0 = c * (3 * _CH) + g * (3 * _GRP)
            outward = plsc.load_gather(xv, [lanes3 + b0])
            xs = plsc.load_gather(xv, [lanes3 + (b0 + 1)])
            ys = plsc.load_gather(xv, [lanes3 + (b0 + 2)])
            idx = 2 * ((xs + _MAX_SHIFT) * _NUM_XS + (ys + _MAX_SHIFT)) + outward
            idx_v[pl.ds(c * _CH + g * _GRP, _GRP)] = idx

    def start_gather(b, c):
        idx_sl = idx_v.at[pl.ds(c * _CH, _CH)]
        pltpu.async_copy(table_v.at[idx_sl], rows.at[b], gsem.at[b])

    def wait_gather(b):
        pltpu.make_async_copy(
            out_ref.at[pl.ds(0, _CH)], rows.at[b], gsem.at[b]).wait()

    def start_store(b, c):
        pltpu.async_copy(
            rows.at[b], out_ref.at[pl.ds(ebase + c * _CH, _CH)], ssem.at[b])

    def wait_store(b):
        pltpu.make_async_copy(
            rows.at[b], out_ref.at[pl.ds(0, _CH)], ssem.at[b]).wait()

    def ch_body(p, carry):
        for b in range(_NBUF):
            c = p * _NBUF + b

            @pl.when(c + _NBUF < _NCH)
            def _():
                compute_idx_chunk(c + _NBUF)

            wait_gather(b)
            start_store(b, c)

            @pl.when(c + _NBUF < _NCH)
            def _():
                wait_store(b)
                start_gather(b, c + _NBUF)

        return carry

    for b in range(_NBUF):
        compute_idx_chunk(b)
        start_gather(b, b)
    lax.fori_loop(0, _NCH // _NBUF, ch_body, 0)
    for b in range(_NBUF):
        wait_store(b)


def kernel(x, emb):
    mesh = plsc.VectorSubcoreMesh(core_axis_name="c", subcore_axis_name="s")
    f = pl.kernel(
        _body,
        out_type=jax.ShapeDtypeStruct((_N, _D), jnp.float32),
        mesh=mesh,
        compiler_params=pltpu.CompilerParams(needs_layout_passes=False),
        scratch_types=[
            pltpu.VMEM((_BPW * 3,), jnp.int32),
            pltpu.VMEM((_BPW,), jnp.int32),
            pltpu.VMEM_SHARED((_NEMB, _D), jnp.float32),
            pltpu.VMEM((_NBUF, _CH, _D), jnp.float32),
            pltpu.SemaphoreType.DMA((_NBUF,)),
            pltpu.SemaphoreType.DMA((_NBUF,)),
        ],
    )
    return f(x.reshape(-1), emb)
